# baseline (speedup 1.0000x reference)
import jax
import jax.numpy as jnp
from jax import lax
from jax.experimental import pallas as pl
from jax.experimental.pallas import tpu as pltpu

N = 4096
IN_F = 512
OUT_F = 256
H = 4
HID = OUT_F // H

BM = 512
BI = 1024
NPROJ = N // BM
NATT = N // BI


def _fused_kernel(vert_ref, w_ref, acomb_ref, out_ref,
                  g_s, ss_s, ssr_s, tmax_s, cmat_s, pq_s, qgtot_s, qtot_s,
                  mask_s):
    k = pl.program_id(0)

    @pl.when(k < NPROJ)
    def _proj():
        g = jnp.dot(vert_ref[...], w_ref[...],
                    preferred_element_type=jnp.float32)
        ss_row = lax.dot_general(
            acomb_ref[...], g, (((0,), (1,)), ((), ())),
            preferred_element_type=jnp.float32)
        ss = jnp.dot(g, acomb_ref[...],
                     preferred_element_type=jnp.float32)
        g_s[pl.ds(k * BM, BM), :] = g
        ss_s[pl.ds(k * BM, BM), :] = ss
        ssr_s[k] = ss_row
        m = jnp.max(ss, axis=0, keepdims=True)

        @pl.when(k == 0)
        def _():
            tmax_s[...] = m

        @pl.when(k > 0)
        def _():
            tmax_s[...] = jnp.maximum(tmax_s[...], m)

    @pl.when(k == NPROJ)
    def _prep():
        c_row = tmax_s[0:1, H:2 * H]
        t_all = ss_s[:, H:2 * H]
        p_all = jnp.exp(t_all - c_row)
        q_all = jnp.exp(0.2 * (t_all - c_row))
        pq_s[:, 0:H] = p_all.astype(jnp.bfloat16)
        pq_s[:, H:2 * H] = q_all.astype(jnp.bfloat16)
        qtot_s[...] = jnp.sum(q_all, axis=0, keepdims=True)
        for h in range(H):
            ghb = g_s[:, h * HID:(h + 1) * HID].astype(jnp.bfloat16)
            qgb = q_all[:, h:h + 1].astype(jnp.bfloat16) * ghb
            cmat_s[:, h * 2 * HID:h * 2 * HID + HID] = (
                p_all[:, h:h + 1].astype(jnp.bfloat16) * ghb)
            cmat_s[:, h * 2 * HID + HID:(h + 1) * 2 * HID] = qgb
            qgtot_s[h:h + 1, :] = jnp.sum(qgb, axis=0, keepdims=True,
                                          dtype=jnp.float32)

    @pl.when(k >= NPROJ)
    def _attn():
        i0 = (k - NPROJ) * BI
        for h in range(H):
            c = tmax_s[0, H + h]
            s_col = ss_s[pl.ds(i0, BI), h:h + 1]
            x = s_col + c
            a_scl = jnp.exp(0.8 * jnp.minimum(x, 0.0))
            b_scl = jnp.exp(-0.8 * jnp.maximum(x, 0.0))
            ns_col_b = (-s_col).astype(jnp.bfloat16)

            for jc in range(NPROJ):
                tb = ssr_s[jc, H + h:H + h + 1, :].astype(jnp.bfloat16)
                mask_s[:, jc * BM:(jc + 1) * BM] = jnp.where(
                    tb > ns_col_b, jnp.bfloat16(1.0), jnp.bfloat16(0.0))
            acc = jnp.dot(
                mask_s[...], cmat_s[:, h * 2 * HID:(h + 1) * 2 * HID],
                preferred_element_type=jnp.float32)
            accpq = jnp.dot(mask_s[...], pq_s[...],
                            preferred_element_type=jnp.float32)

            numer = a_scl * acc[:, :HID] + b_scl * (qgtot_s[h:h + 1, :]
                                                    - acc[:, HID:])
            denom = a_scl * accpq[:, h:h + 1] + b_scl * (
                qtot_s[:, h:h + 1] - accpq[:, H + h:H + h + 1])
            o = numer / denom
            out_ref[:, h * HID:(h + 1) * HID] = jnp.where(
                o > 0.0, o, jnp.exp(jnp.minimum(o, 0.0)) - 1.0)


def kernel(vert, W, a_src, a_dst):
    idx = jnp.arange(OUT_F)
    head = idx // HID
    sel = (head[:, None] == jnp.arange(H)[None, :]).astype(jnp.float32)
    acomb = jnp.concatenate(
        [sel * a_src.reshape(-1)[:, None], sel * a_dst.reshape(-1)[:, None]],
        axis=1)

    out = pl.pallas_call(
        _fused_kernel,
        grid=(NPROJ + NATT,),
        in_specs=[
            pl.BlockSpec((BM, IN_F), lambda k: (jnp.minimum(k, NPROJ - 1), 0)),
            pl.BlockSpec((IN_F, OUT_F), lambda k: (0, 0)),
            pl.BlockSpec((OUT_F, 2 * H), lambda k: (0, 0)),
        ],
        out_specs=pl.BlockSpec(
            (BI, OUT_F), lambda k: (jnp.maximum(k - NPROJ, 0), 0)),
        out_shape=jax.ShapeDtypeStruct((N, OUT_F), jnp.float32),
        scratch_shapes=[
            pltpu.VMEM((N, OUT_F), jnp.float32),
            pltpu.VMEM((N, 2 * H), jnp.float32),
            pltpu.VMEM((NPROJ, 2 * H, BM), jnp.float32),
            pltpu.VMEM((1, 2 * H), jnp.float32),
            pltpu.VMEM((N, 2 * OUT_F), jnp.bfloat16),
            pltpu.VMEM((N, 2 * H), jnp.bfloat16),
            pltpu.VMEM((H, HID), jnp.float32),
            pltpu.VMEM((1, H), jnp.float32),
            pltpu.VMEM((BI, N), jnp.bfloat16),
        ],
    )(vert, W, acomb)
    return out
